# Initial kernel scaffold; baseline (speedup 1.0000x reference)
import jax
import jax.numpy as jnp
from jax.experimental import pallas as pl


def kernel(x, edge_index, action, aconv_W, aconv_b, a_W1, a_b1, a_W2, a_b2, a_W3, a_b3, cconv_W, cconv_b, c_W1, c_b1, c_W2, c_b2, c_W3, c_b3, actor_logstd):
    raise NotImplementedError("write your pallas kernel here")


# speedup vs baseline: 4.2843x; 4.2843x over previous
import functools

import jax
import jax.numpy as jnp
from jax import lax
from jax.experimental import pallas as pl
from jax.experimental.pallas import tpu as pltpu

N = 10000
D = 128
H = 64
A = 7
E = 320000


def _tail_kernel(T_ref, xs_ref, dinv_ref, Wcat_ref, bcat_ref,
                 aW1_ref, ab1_ref, aW2_ref, ab2_ref, aW3_ref, ab3_ref,
                 cW1_ref, cb1_ref, cW2_ref, cb2_ref, cW3_ref, cb3_ref,
                 action_ref, logstd_ref,
                 logprob_ref, entropy_ref, value_ref,
                 zc_scratch):
    z = (T_ref[...] + xs_ref[...]) * dinv_ref[...]
    zc = jnp.dot(z, Wcat_ref[...], preferred_element_type=jnp.float32)
    zc = zc + bcat_ref[...]
    b = jax.lax.bitcast_convert_type(zc, jnp.int32)
    neg_mask = b >> 31
    ub = jax.lax.bitcast_convert_type(b, jnp.uint32)
    xor_val = jax.lax.bitcast_convert_type(neg_mask, jnp.uint32) | jnp.uint32(0x80000000)
    zc_scratch[...] = ub ^ xor_val
    zcu = zc_scratch[...]

    def body(i, carry):
        loA, hiA, loB, hiB = carry
        midA = loA + (hiA - loA) // jnp.uint32(2)
        midB = loB + (hiB - loB) // jnp.uint32(2)
        cA = jnp.sum((zcu <= midA).astype(jnp.int32), axis=0, keepdims=True)
        cB = jnp.sum((zcu <= midB).astype(jnp.int32), axis=0, keepdims=True)
        geA = cA >= 5000
        geB = cB >= 5001
        hiA = jnp.where(geA, midA, hiA)
        loA = jnp.where(geA, loA, midA + jnp.uint32(1))
        hiB = jnp.where(geB, midB, hiB)
        loB = jnp.where(geB, loB, midB + jnp.uint32(1))
        return loA, hiA, loB, hiB

    lo0 = jnp.zeros((1, 2 * D), dtype=jnp.uint32)
    hi0 = jnp.full((1, 2 * D), 0xFFFFFFFF, dtype=jnp.uint32)
    loA, _, loB, _ = lax.fori_loop(0, 32, body, (lo0, hi0, lo0, hi0))

    def u2f(u):
        is_pos = u >= jnp.uint32(0x80000000)
        ub2 = jnp.where(is_pos, u ^ jnp.uint32(0x80000000), ~u)
        return jax.lax.bitcast_convert_type(ub2, jnp.float32)

    med = 0.5 * (u2f(loA) + u2f(loB))
    med2 = med.reshape(2, D)
    ma = med2[0:1, :]
    mc = med2[1:2, :]

    t = jnp.tanh(jnp.dot(ma, aW1_ref[...], preferred_element_type=jnp.float32) + ab1_ref[...])
    t = jnp.tanh(jnp.dot(t, aW2_ref[...], preferred_element_type=jnp.float32) + ab2_ref[...])
    action_mean = jnp.dot(t, aW3_ref[...], preferred_element_type=jnp.float32) + ab3_ref[...]

    logstd = logstd_ref[...]
    action = action_ref[...]
    action_std = jnp.exp(logstd)
    lp = (-((action - action_mean) ** 2) / (2.0 * action_std ** 2)
          - logstd - 0.5 * jnp.log(2.0 * jnp.pi))
    logprob_ref[...] = jnp.sum(lp, axis=1, keepdims=True)
    ent = 0.5 + 0.5 * jnp.log(2.0 * jnp.pi) + logstd
    entropy_ref[...] = jnp.sum(ent, axis=1, keepdims=True)

    tc = jnp.tanh(jnp.dot(mc, cW1_ref[...], preferred_element_type=jnp.float32) + cb1_ref[...])
    tc = jnp.tanh(jnp.dot(tc, cW2_ref[...], preferred_element_type=jnp.float32) + cb2_ref[...])
    value_ref[...] = jnp.dot(tc, cW3_ref[...], preferred_element_type=jnp.float32) + cb3_ref[...]


def _tail(T, xs, dinv, Wcat, bcat,
          a_W1, a_b1, a_W2, a_b2, a_W3, a_b3,
          c_W1, c_b1, c_W2, c_b2, c_W3, c_b3,
          action, actor_logstd):
    out_shapes = (
        jax.ShapeDtypeStruct((1, 1), jnp.float32),
        jax.ShapeDtypeStruct((1, 1), jnp.float32),
        jax.ShapeDtypeStruct((1, 1), jnp.float32),
    )
    return pl.pallas_call(
        _tail_kernel,
        out_shape=out_shapes,
        scratch_shapes=[pltpu.VMEM((N, 2 * D), jnp.uint32)],
    )(T, xs, dinv, Wcat, bcat,
      a_W1, a_b1.reshape(1, H), a_W2, a_b2.reshape(1, H), a_W3, a_b3.reshape(1, A),
      c_W1, c_b1.reshape(1, H), c_W2, c_b2.reshape(1, H), c_W3, c_b3.reshape(1, 1),
      action, actor_logstd)


def kernel(x, edge_index, action, aconv_W, aconv_b, a_W1, a_b1, a_W2, a_b2,
           a_W3, a_b3, cconv_W, cconv_b, c_W1, c_b1, c_W2, c_b2, c_W3, c_b3,
           actor_logstd):
    row = edge_index[0]
    col = edge_index[1]
    deg = jnp.zeros((N,), jnp.float32).at[col].add(1.0) + 1.0
    dinv = lax.rsqrt(deg)
    xs = x * dinv[:, None]
    T = jnp.zeros((N, D), jnp.float32).at[col].add(jnp.take(xs, row, axis=0))

    Wcat = jnp.concatenate([aconv_W, cconv_W], axis=1)
    bcat = jnp.concatenate([aconv_b, cconv_b], axis=0).reshape(1, 2 * D)

    log_prob, entropy, value = _tail(
        T, xs, dinv.reshape(N, 1), Wcat, bcat,
        a_W1, a_b1, a_W2, a_b2, a_W3, a_b3,
        c_W1, c_b1, c_W2, c_b2, c_W3, c_b3,
        action, actor_logstd)
    return (action, log_prob.reshape(1), entropy.reshape(1), value)


# speedup vs baseline: 30.5401x; 7.1284x over previous
import jax
import jax.numpy as jnp
from jax import lax
from jax.experimental import pallas as pl
from jax.experimental.pallas import tpu as pltpu
from jax.experimental.pallas import tpu_sc as plsc

N = 10000
D = 128
H = 64
A = 7
E = 320000

NC = 2
NS = 16
NW = NC * NS
CH = 125
NCH = 80
EPW = E // NW
NPAD = 10240
DEG_PT = NPAD // NS
ROWS_PT = N // NS

_MESH = dict(core_axis_name="c", subcore_axis_name="s", num_cores=NC,
             num_subcores=NS)
_SC_PARAMS = pltpu.CompilerParams(use_tc_tiling_on_sc=False)


def _deg_body(col_hbm, out_hbm, col_slab, ones_v, zbuf, deg_acc):
    cc = lax.axis_index("c")
    sid = lax.axis_index("s")
    wid = cc * NS + sid

    @pl.loop(0, 8)
    def _(i):
        ones_v[pl.ds(i * 16, 16)] = jnp.ones((16,), jnp.float32)

    @pl.loop(0, DEG_PT // 16)
    def _(i):
        zbuf[pl.ds(i * 16, 16)] = jnp.zeros((16,), jnp.float32)

    pltpu.sync_copy(zbuf, deg_acc.at[pl.ds(sid * DEG_PT, DEG_PT)])
    pltpu.sync_copy(col_hbm.at[wid], col_slab)
    plsc.subcore_barrier()

    @pl.loop(0, NCH)
    def _(j):
        pltpu.sync_copy(ones_v.at[pl.ds(0, CH)],
                        deg_acc.at[col_slab.at[j]], add=True)

    plsc.subcore_barrier()
    pltpu.sync_copy(deg_acc.at[pl.ds(sid * DEG_PT, DEG_PT)],
                    out_hbm.at[cc, pl.ds(sid * DEG_PT, DEG_PT)])


def _deg_call(col3):
    return pl.kernel(
        _deg_body,
        out_type=jax.ShapeDtypeStruct((NC, NPAD), jnp.float32),
        mesh=plsc.VectorSubcoreMesh(**_MESH),
        compiler_params=_SC_PARAMS,
        scratch_types=[
            pltpu.VMEM((NCH, CH), jnp.int32),
            pltpu.VMEM((128,), jnp.float32),
            pltpu.VMEM((DEG_PT,), jnp.float32),
            pltpu.VMEM_SHARED((NPAD,), jnp.float32),
        ],
    )(col3)


def _agg_body(xs_hbm, row_hbm, col_hbm, out_hbm, row_slab, col_slab, gbuf,
              t_acc):
    cc = lax.axis_index("c")
    sid = lax.axis_index("s")
    wid = cc * NS + sid

    @pl.loop(0, CH)
    def _(r):
        @pl.loop(0, D // 16)
        def _(q):
            gbuf[r, pl.ds(q * 16, 16)] = jnp.zeros((16,), jnp.float32)

    @pl.loop(0, ROWS_PT // CH)
    def _(k):
        pltpu.sync_copy(gbuf, t_acc.at[pl.ds(sid * ROWS_PT + k * CH, CH)])

    pltpu.sync_copy(row_hbm.at[wid], row_slab)
    pltpu.sync_copy(col_hbm.at[wid], col_slab)
    plsc.subcore_barrier()

    @pl.loop(0, NCH)
    def _(j):
        pltpu.sync_copy(xs_hbm.at[row_slab.at[j]], gbuf)
        pltpu.sync_copy(gbuf, t_acc.at[col_slab.at[j]], add=True)

    plsc.subcore_barrier()

    @pl.loop(0, ROWS_PT // CH)
    def _(k):
        off = sid * ROWS_PT + k * CH
        pltpu.sync_copy(t_acc.at[pl.ds(off, CH)], gbuf)
        pltpu.sync_copy(gbuf, out_hbm.at[cc, pl.ds(off, CH)])


def _agg_call(xs, row3, col3):
    return pl.kernel(
        _agg_body,
        out_type=jax.ShapeDtypeStruct((NC, N, D), jnp.float32),
        mesh=plsc.VectorSubcoreMesh(**_MESH),
        compiler_params=_SC_PARAMS,
        scratch_types=[
            pltpu.VMEM((NCH, CH), jnp.int32),
            pltpu.VMEM((NCH, CH), jnp.int32),
            pltpu.VMEM((CH, D), jnp.float32),
            pltpu.VMEM_SHARED((N, D), jnp.float32),
        ],
    )(xs, row3, col3)


def _scale_kernel(d0_ref, d1_ref, x_ref, xs_ref, dinv_ref):
    dinv = lax.rsqrt(d0_ref[...] + d1_ref[...] + 1.0)
    dinv_ref[...] = dinv
    xs_ref[...] = x_ref[...] * dinv


def _scale_call(deg0, deg1, x):
    return pl.pallas_call(
        _scale_kernel,
        out_shape=(
            jax.ShapeDtypeStruct((N, D), jnp.float32),
            jax.ShapeDtypeStruct((N, 1), jnp.float32),
        ),
    )(deg0, deg1, x)


def _tail_kernel(T0_ref, T1_ref, xs_ref, dinv_ref, Wcat_ref, bcat_ref,
                 aW1_ref, ab1_ref, aW2_ref, ab2_ref, aW3_ref, ab3_ref,
                 cW1_ref, cb1_ref, cW2_ref, cb2_ref, cW3_ref, cb3_ref,
                 action_ref, logstd_ref,
                 logprob_ref, entropy_ref, value_ref,
                 zc_scratch):
    z = (T0_ref[...] + T1_ref[...] + xs_ref[...]) * dinv_ref[...]
    zc = jnp.dot(z, Wcat_ref[...], preferred_element_type=jnp.float32)
    zc = zc + bcat_ref[...]
    b = jax.lax.bitcast_convert_type(zc, jnp.int32)
    neg_mask = b >> 31
    ub = jax.lax.bitcast_convert_type(b, jnp.uint32)
    xor_val = jax.lax.bitcast_convert_type(neg_mask, jnp.uint32) | jnp.uint32(0x80000000)
    zc_scratch[...] = ub ^ xor_val
    zcu = zc_scratch[...]

    def body(i, carry):
        loA, hiA, loB, hiB = carry
        midA = loA + (hiA - loA) // jnp.uint32(2)
        midB = loB + (hiB - loB) // jnp.uint32(2)
        cA = jnp.sum((zcu <= midA).astype(jnp.int32), axis=0, keepdims=True)
        cB = jnp.sum((zcu <= midB).astype(jnp.int32), axis=0, keepdims=True)
        geA = cA >= 5000
        geB = cB >= 5001
        hiA = jnp.where(geA, midA, hiA)
        loA = jnp.where(geA, loA, midA + jnp.uint32(1))
        hiB = jnp.where(geB, midB, hiB)
        loB = jnp.where(geB, loB, midB + jnp.uint32(1))
        return loA, hiA, loB, hiB

    lo0 = jnp.zeros((1, 2 * D), dtype=jnp.uint32)
    hi0 = jnp.full((1, 2 * D), 0xFFFFFFFF, dtype=jnp.uint32)
    loA, _, loB, _ = lax.fori_loop(0, 32, body, (lo0, hi0, lo0, hi0))

    def u2f(u):
        is_pos = u >= jnp.uint32(0x80000000)
        ub2 = jnp.where(is_pos, u ^ jnp.uint32(0x80000000), ~u)
        return jax.lax.bitcast_convert_type(ub2, jnp.float32)

    med = 0.5 * (u2f(loA) + u2f(loB))
    med2 = med.reshape(2, D)
    ma = med2[0:1, :]
    mc = med2[1:2, :]

    t = jnp.tanh(jnp.dot(ma, aW1_ref[...], preferred_element_type=jnp.float32) + ab1_ref[...])
    t = jnp.tanh(jnp.dot(t, aW2_ref[...], preferred_element_type=jnp.float32) + ab2_ref[...])
    action_mean = jnp.dot(t, aW3_ref[...], preferred_element_type=jnp.float32) + ab3_ref[...]

    logstd = logstd_ref[...]
    action = action_ref[...]
    action_std = jnp.exp(logstd)
    lp = (-((action - action_mean) ** 2) / (2.0 * action_std ** 2)
          - logstd - 0.5 * jnp.log(2.0 * jnp.pi))
    logprob_ref[...] = jnp.sum(lp, axis=1, keepdims=True)
    ent = 0.5 + 0.5 * jnp.log(2.0 * jnp.pi) + logstd
    entropy_ref[...] = jnp.sum(ent, axis=1, keepdims=True)

    tc = jnp.tanh(jnp.dot(mc, cW1_ref[...], preferred_element_type=jnp.float32) + cb1_ref[...])
    tc = jnp.tanh(jnp.dot(tc, cW2_ref[...], preferred_element_type=jnp.float32) + cb2_ref[...])
    value_ref[...] = jnp.dot(tc, cW3_ref[...], preferred_element_type=jnp.float32) + cb3_ref[...]


def _tail(T0, T1, xs, dinv, Wcat, bcat,
          a_W1, a_b1, a_W2, a_b2, a_W3, a_b3,
          c_W1, c_b1, c_W2, c_b2, c_W3, c_b3,
          action, actor_logstd):
    out_shapes = (
        jax.ShapeDtypeStruct((1, 1), jnp.float32),
        jax.ShapeDtypeStruct((1, 1), jnp.float32),
        jax.ShapeDtypeStruct((1, 1), jnp.float32),
    )
    return pl.pallas_call(
        _tail_kernel,
        out_shape=out_shapes,
        scratch_shapes=[pltpu.VMEM((N, 2 * D), jnp.uint32)],
    )(T0, T1, xs, dinv, Wcat, bcat,
      a_W1, a_b1.reshape(1, H), a_W2, a_b2.reshape(1, H), a_W3, a_b3.reshape(1, A),
      c_W1, c_b1.reshape(1, H), c_W2, c_b2.reshape(1, H), c_W3, c_b3.reshape(1, 1),
      action, actor_logstd)


def kernel(x, edge_index, action, aconv_W, aconv_b, a_W1, a_b1, a_W2, a_b2,
           a_W3, a_b3, cconv_W, cconv_b, c_W1, c_b1, c_W2, c_b2, c_W3, c_b3,
           actor_logstd):
    row3 = edge_index[0].reshape(NW, NCH, CH)
    col3 = edge_index[1].reshape(NW, NCH, CH)

    degp = _deg_call(col3)
    deg0 = degp[0, :N].reshape(N, 1)
    deg1 = degp[1, :N].reshape(N, 1)
    xs, dinv = _scale_call(deg0, deg1, x)
    Tp = _agg_call(xs, row3, col3)

    Wcat = jnp.concatenate([aconv_W, cconv_W], axis=1)
    bcat = jnp.concatenate([aconv_b, cconv_b], axis=0).reshape(1, 2 * D)

    log_prob, entropy, value = _tail(
        Tp[0], Tp[1], xs, dinv, Wcat, bcat,
        a_W1, a_b1, a_W2, a_b2, a_W3, a_b3,
        c_W1, c_b1, c_W2, c_b2, c_W3, c_b3,
        action, actor_logstd)
    return (action, log_prob.reshape(1), entropy.reshape(1), value)


# speedup vs baseline: 40.5300x; 1.3271x over previous
import jax
import jax.numpy as jnp
from jax import lax
from jax.experimental import pallas as pl
from jax.experimental.pallas import tpu as pltpu
from jax.experimental.pallas import tpu_sc as plsc

N = 10000
D = 128
H = 64
A = 7
E = 320000

NC = 2
NS = 16
NW = NC * NS
CH = 125
NCH = 80
CPS = 20
EPW = E // NW
NPAD = 10240
DEG_PT = NPAD // NS
ROWS_PT = N // NS

_MESH = dict(core_axis_name="c", subcore_axis_name="s", num_cores=NC,
             num_subcores=NS)
_SC_PARAMS = pltpu.CompilerParams(use_tc_tiling_on_sc=False)


def _deg_body(col_hbm, out_hbm, col_slab, ones_v, zbuf, deg_acc):
    cc = lax.axis_index("c")
    sid = lax.axis_index("s")
    wid = cc * NS + sid

    @pl.loop(0, 8)
    def _(i):
        ones_v[pl.ds(i * 16, 16)] = jnp.ones((16,), jnp.float32)

    @pl.loop(0, DEG_PT // 16)
    def _(i):
        zbuf[pl.ds(i * 16, 16)] = jnp.zeros((16,), jnp.float32)

    pltpu.sync_copy(zbuf, deg_acc.at[pl.ds(sid * DEG_PT, DEG_PT)])
    pltpu.sync_copy(col_hbm.at[wid], col_slab)
    plsc.subcore_barrier()

    @pl.loop(0, NCH)
    def _(j):
        pltpu.sync_copy(ones_v.at[pl.ds(0, CH)],
                        deg_acc.at[col_slab.at[j]], add=True)

    plsc.subcore_barrier()
    pltpu.sync_copy(deg_acc.at[pl.ds(sid * DEG_PT, DEG_PT)],
                    out_hbm.at[cc, pl.ds(sid * DEG_PT, DEG_PT)])


def _deg_call(col3):
    return pl.kernel(
        _deg_body,
        out_type=jax.ShapeDtypeStruct((NC, NPAD), jnp.float32),
        mesh=plsc.VectorSubcoreMesh(**_MESH),
        compiler_params=_SC_PARAMS,
        scratch_types=[
            pltpu.VMEM((NCH, CH), jnp.int32),
            pltpu.VMEM((128,), jnp.float32),
            pltpu.VMEM((DEG_PT,), jnp.float32),
            pltpu.VMEM_SHARED((NPAD,), jnp.float32),
        ],
    )(col3)


def _agg_body(xs_hbm, row_hbm, col_hbm, out_hbm, row_slab, col_slab, gbuf,
              t_acc, sem0, sem1):
    cc = lax.axis_index("c")
    sid = lax.axis_index("s")
    wid = cc * NS + sid
    sems = (sem0, sem1)
    g0 = gbuf.at[0]

    @pl.loop(0, CH)
    def _(r):
        @pl.loop(0, D // 16)
        def _(q):
            g0[r, pl.ds(q * 16, 16)] = jnp.zeros((16,), jnp.float32)

    @pl.loop(0, ROWS_PT // CH)
    def _(k):
        pltpu.sync_copy(g0, t_acc.at[pl.ds(sid * ROWS_PT + k * CH, CH)])

    plsc.subcore_barrier()

    @pl.loop(0, NCH // CPS)
    def _(s):
        pltpu.sync_copy(row_hbm.at[wid, pl.ds(s * CPS, CPS)], row_slab)
        pltpu.sync_copy(col_hbm.at[wid, pl.ds(s * CPS, CPS)], col_slab)
        for b in range(2):
            pltpu.async_copy(xs_hbm.at[row_slab.at[b]], gbuf.at[b], sems[b])

        @pl.loop(0, CPS // 2)
        def _(i):
            for b in range(2):
                j = 2 * i + b
                pltpu.make_async_copy(xs_hbm.at[row_slab.at[j]], gbuf.at[b],
                                      sems[b]).wait()
                pltpu.sync_copy(gbuf.at[b], t_acc.at[col_slab.at[j]], add=True)

                @pl.when(j + 2 < CPS)
                def _():
                    pltpu.async_copy(xs_hbm.at[row_slab.at[j + 2]],
                                     gbuf.at[b], sems[b])

    plsc.subcore_barrier()

    @pl.loop(0, ROWS_PT // CH)
    def _(k):
        off = sid * ROWS_PT + k * CH
        pltpu.sync_copy(t_acc.at[pl.ds(off, CH)], g0)
        pltpu.sync_copy(g0, out_hbm.at[cc, pl.ds(off, CH)])


def _agg_call(xs, row3, col3):
    return pl.kernel(
        _agg_body,
        out_type=jax.ShapeDtypeStruct((NC, N, D), jnp.float32),
        mesh=plsc.VectorSubcoreMesh(**_MESH),
        compiler_params=_SC_PARAMS,
        scratch_types=[
            pltpu.VMEM((CPS, CH), jnp.int32),
            pltpu.VMEM((CPS, CH), jnp.int32),
            pltpu.VMEM((2, CH, D), jnp.float32),
            pltpu.VMEM_SHARED((N, D), jnp.float32),
            pltpu.SemaphoreType.DMA,
            pltpu.SemaphoreType.DMA,
        ],
    )(xs, row3, col3)


def _scale_kernel(d0_ref, d1_ref, x_ref, xs_ref, dinv_ref):
    dinv = lax.rsqrt(d0_ref[...] + d1_ref[...] + 1.0)
    dinv_ref[...] = dinv
    xs_ref[...] = x_ref[...] * dinv


def _scale_call(deg0, deg1, x):
    return pl.pallas_call(
        _scale_kernel,
        out_shape=(
            jax.ShapeDtypeStruct((N, D), jnp.float32),
            jax.ShapeDtypeStruct((N, 1), jnp.float32),
        ),
    )(deg0, deg1, x)


def _tail_kernel(T0_ref, T1_ref, xs_ref, dinv_ref, Wcat_ref, bcat_ref,
                 aW1_ref, ab1_ref, aW2_ref, ab2_ref, aW3_ref, ab3_ref,
                 cW1_ref, cb1_ref, cW2_ref, cb2_ref, cW3_ref, cb3_ref,
                 action_ref, logstd_ref,
                 logprob_ref, entropy_ref, value_ref,
                 zc_scratch):
    z = (T0_ref[...] + T1_ref[...] + xs_ref[...]) * dinv_ref[...]
    zc = jnp.dot(z, Wcat_ref[...], preferred_element_type=jnp.float32)
    zc = zc + bcat_ref[...]
    b = jax.lax.bitcast_convert_type(zc, jnp.int32)
    neg_mask = b >> 31
    ub = jax.lax.bitcast_convert_type(b, jnp.uint32)
    xor_val = jax.lax.bitcast_convert_type(neg_mask, jnp.uint32) | jnp.uint32(0x80000000)
    zc_scratch[...] = ub ^ xor_val
    zcu = zc_scratch[...]

    def body(i, carry):
        loA, hiA = carry
        midA = loA + (hiA - loA) // jnp.uint32(2)
        cA = jnp.sum((zcu <= midA).astype(jnp.int32), axis=0, keepdims=True)
        geA = cA >= 5000
        hiA = jnp.where(geA, midA, hiA)
        loA = jnp.where(geA, loA, midA + jnp.uint32(1))
        return loA, hiA

    lo0 = jnp.zeros((1, 2 * D), dtype=jnp.uint32)
    hi0 = jnp.full((1, 2 * D), 0xFFFFFFFF, dtype=jnp.uint32)
    loA, _ = lax.fori_loop(0, 32, body, (lo0, hi0))

    cnt = jnp.sum((zcu <= loA).astype(jnp.int32), axis=0, keepdims=True)
    zcs = jax.lax.bitcast_convert_type(zcu ^ jnp.uint32(0x80000000), jnp.int32)
    loS = jax.lax.bitcast_convert_type(loA ^ jnp.uint32(0x80000000), jnp.int32)
    big = jnp.int32(0x7FFFFFFF)
    above = jnp.where(zcs > loS, zcs, big)
    minS = jnp.min(above, axis=0, keepdims=True)
    minU = jax.lax.bitcast_convert_type(minS, jnp.uint32) ^ jnp.uint32(0x80000000)
    loB = jnp.where(cnt >= 5001, loA, minU)

    def u2f(u):
        is_pos = u >= jnp.uint32(0x80000000)
        ub2 = jnp.where(is_pos, u ^ jnp.uint32(0x80000000), ~u)
        return jax.lax.bitcast_convert_type(ub2, jnp.float32)

    med = 0.5 * (u2f(loA) + u2f(loB))
    med2 = med.reshape(2, D)
    ma = med2[0:1, :]
    mc = med2[1:2, :]

    t = jnp.tanh(jnp.dot(ma, aW1_ref[...], preferred_element_type=jnp.float32) + ab1_ref[...])
    t = jnp.tanh(jnp.dot(t, aW2_ref[...], preferred_element_type=jnp.float32) + ab2_ref[...])
    action_mean = jnp.dot(t, aW3_ref[...], preferred_element_type=jnp.float32) + ab3_ref[...]

    logstd = logstd_ref[...]
    action = action_ref[...]
    action_std = jnp.exp(logstd)
    lp = (-((action - action_mean) ** 2) / (2.0 * action_std ** 2)
          - logstd - 0.5 * jnp.log(2.0 * jnp.pi))
    logprob_ref[...] = jnp.sum(lp, axis=1, keepdims=True)
    ent = 0.5 + 0.5 * jnp.log(2.0 * jnp.pi) + logstd
    entropy_ref[...] = jnp.sum(ent, axis=1, keepdims=True)

    tc = jnp.tanh(jnp.dot(mc, cW1_ref[...], preferred_element_type=jnp.float32) + cb1_ref[...])
    tc = jnp.tanh(jnp.dot(tc, cW2_ref[...], preferred_element_type=jnp.float32) + cb2_ref[...])
    value_ref[...] = jnp.dot(tc, cW3_ref[...], preferred_element_type=jnp.float32) + cb3_ref[...]


def _tail(T0, T1, xs, dinv, Wcat, bcat,
          a_W1, a_b1, a_W2, a_b2, a_W3, a_b3,
          c_W1, c_b1, c_W2, c_b2, c_W3, c_b3,
          action, actor_logstd):
    out_shapes = (
        jax.ShapeDtypeStruct((1, 1), jnp.float32),
        jax.ShapeDtypeStruct((1, 1), jnp.float32),
        jax.ShapeDtypeStruct((1, 1), jnp.float32),
    )
    return pl.pallas_call(
        _tail_kernel,
        out_shape=out_shapes,
        scratch_shapes=[pltpu.VMEM((N, 2 * D), jnp.uint32)],
    )(T0, T1, xs, dinv, Wcat, bcat,
      a_W1, a_b1.reshape(1, H), a_W2, a_b2.reshape(1, H), a_W3, a_b3.reshape(1, A),
      c_W1, c_b1.reshape(1, H), c_W2, c_b2.reshape(1, H), c_W3, c_b3.reshape(1, 1),
      action, actor_logstd)


def kernel(x, edge_index, action, aconv_W, aconv_b, a_W1, a_b1, a_W2, a_b2,
           a_W3, a_b3, cconv_W, cconv_b, c_W1, c_b1, c_W2, c_b2, c_W3, c_b3,
           actor_logstd):
    row3 = edge_index[0].reshape(NW, NCH, CH)
    col3 = edge_index[1].reshape(NW, NCH, CH)

    degp = _deg_call(col3)
    deg0 = degp[0, :N].reshape(N, 1)
    deg1 = degp[1, :N].reshape(N, 1)
    xs, dinv = _scale_call(deg0, deg1, x)
    Tp = _agg_call(xs, row3, col3)

    Wcat = jnp.concatenate([aconv_W, cconv_W], axis=1)
    bcat = jnp.concatenate([aconv_b, cconv_b], axis=0).reshape(1, 2 * D)

    log_prob, entropy, value = _tail(
        Tp[0], Tp[1], xs, dinv, Wcat, bcat,
        a_W1, a_b1, a_W2, a_b2, a_W3, a_b3,
        c_W1, c_b1, c_W2, c_b2, c_W3, c_b3,
        action, actor_logstd)
    return (action, log_prob.reshape(1), entropy.reshape(1), value)
